# baseline (speedup 1.0000x reference)
import jax
import jax.numpy as jnp
from jax.experimental import pallas as pl

NUM_CANVASES = 3
B, C, H, W = 16, 3, 512, 512

_ROWS = B * C * H // 2
_LANES = 2 * W
_BLOCK_ROWS = 1024


def _copy_kernel(src_ref, dst_ref):
    dst_ref[...] = src_ref[...]


def kernel(img_batch, canvases):
    del canvases
    flat = img_batch.reshape(_ROWS, _LANES)
    out = pl.pallas_call(
        _copy_kernel,
        grid=(_ROWS // _BLOCK_ROWS,),
        in_specs=[pl.BlockSpec((_BLOCK_ROWS, _LANES), lambda i: (i, 0))],
        out_specs=pl.BlockSpec((_BLOCK_ROWS, _LANES), lambda i: (i, 0)),
        out_shape=jax.ShapeDtypeStruct((_ROWS, _LANES), jnp.float32),
    )(flat)
    return out.reshape(B, C, H, W)


# speedup vs baseline: 4.4989x; 4.4989x over previous
import jax
import jax.numpy as jnp
from jax.experimental import pallas as pl

NUM_CANVASES = 3
B, C, H, W = 16, 3, 512, 512

_ROWS = B * C * H
_LANES = W
_BLOCK_ROWS = 4096


def _copy_kernel(src_ref, dst_ref):
    dst_ref[...] = src_ref[...]


def kernel(img_batch, canvases):
    del canvases
    flat = img_batch.reshape(_ROWS, _LANES)
    out = pl.pallas_call(
        _copy_kernel,
        grid=(_ROWS // _BLOCK_ROWS,),
        in_specs=[pl.BlockSpec((_BLOCK_ROWS, _LANES), lambda i: (i, 0))],
        out_specs=pl.BlockSpec((_BLOCK_ROWS, _LANES), lambda i: (i, 0)),
        out_shape=jax.ShapeDtypeStruct((_ROWS, _LANES), jnp.float32),
    )(flat)
    return out.reshape(B, C, H, W)


# speedup vs baseline: 4.5321x; 1.0074x over previous
import jax
import jax.numpy as jnp
from jax.experimental import pallas as pl
from jax.experimental.pallas import tpu as pltpu

NUM_CANVASES = 3
B, C, H, W = 16, 3, 512, 512

_ROWS = B * C * H
_BLOCK_ROWS = 6144


def _copy_kernel(src_ref, dst_ref):
    dst_ref[...] = src_ref[...]


def kernel(img_batch, canvases):
    del canvases
    flat = img_batch.reshape(_ROWS, W)
    out = pl.pallas_call(
        _copy_kernel,
        grid=(_ROWS // _BLOCK_ROWS,),
        in_specs=[pl.BlockSpec((_BLOCK_ROWS, W), lambda i: (i, 0))],
        out_specs=pl.BlockSpec((_BLOCK_ROWS, W), lambda i: (i, 0)),
        out_shape=jax.ShapeDtypeStruct((_ROWS, W), jnp.float32),
        compiler_params=pltpu.CompilerParams(
            dimension_semantics=("arbitrary",),
        ),
    )(flat)
    return out.reshape(B, C, H, W)
